# baseline (speedup 1.0000x reference)
import functools

import jax
import jax.numpy as jnp
from jax import lax
from jax.experimental import pallas as pl
from jax.experimental.pallas import tpu as pltpu
from jax.experimental.pallas import tpu_sc as plsc

B = 4096
L = 200
D = 64
DW = 128
H = 32
NC = 2
NS = 16
NW = NC * NS
BPW = B // NW
NACC = 4
NV = D // 16


def _pool_sc(x, table):
    mesh = plsc.VectorSubcoreMesh(core_axis_name="core", subcore_axis_name="subcore")

    @functools.partial(
        pl.kernel,
        out_type=jax.ShapeDtypeStruct((B, D), jnp.float32),
        mesh=mesh,
        scratch_types=[
            pltpu.VMEM((BPW, L), jnp.int32),
            pltpu.VMEM((L, BPW), jnp.int32),
            pltpu.VMEM((NACC, BPW, DW), jnp.float32),
            pltpu.VMEM((BPW, D), jnp.float32),
        ]
        + [pltpu.SemaphoreType.DMA] * NACC,
        compiler_params=pltpu.CompilerParams(
            use_tc_tiling_on_sc=False, needs_layout_passes=False
        ),
    )
    def pool(x_hbm, table_hbm, out_hbm, idx_raw, idx_v, accs_v, out_v, *sems):
        wid = lax.axis_index("subcore") * NC + lax.axis_index("core")
        base = wid * BPW
        pltpu.sync_copy(x_hbm.at[pl.ds(base, BPW)], idx_raw)

        lanes = lax.iota(jnp.int32, 16)

        @pl.loop(0, L)
        def _(j):
            cols = jnp.zeros((16,), jnp.int32) + j
            for g in range(BPW // 16):
                v = plsc.load_gather(idx_raw, [lanes + 16 * g, cols])
                idx_v[j, pl.ds(16 * g, 16)] = v

        for k in range(NACC):
            pltpu.async_copy(table_hbm.at[idx_v.at[k]], accs_v.at[k], sems[k])

        @pl.loop(NACC, L, step=NACC)
        def _(j):
            for k in range(NACC):
                pltpu.make_async_copy(
                    table_hbm.at[idx_v.at[0]], accs_v.at[k], sems[k]
                ).wait()
                pltpu.async_copy(
                    table_hbm.at[idx_v.at[j + k]], accs_v.at[k], sems[k], add=True
                )

        for k in range(NACC):
            pltpu.make_async_copy(
                table_hbm.at[idx_v.at[0]], accs_v.at[k], sems[k]
            ).wait()

        @pl.loop(0, BPW)
        def _(b):
            for i in range(NV):
                s = pl.ds(16 * i, 16)
                out_v[b, s] = (
                    (accs_v[0, b, s] + accs_v[1, b, s])
                    + (accs_v[2, b, s] + accs_v[3, b, s])
                )

        pltpu.sync_copy(out_v, out_hbm.at[pl.ds(base, BPW)])

    return pool(x, table)


def _mlp_tc(pooled_sum, w1t, b1, w2, b2):
    def body(p_ref, w1_ref, b1_ref, w2_ref, b2_ref, o_ref):
        p = p_ref[...] * (1.0 / L)
        h = jnp.dot(p, w1_ref[...], preferred_element_type=jnp.float32) + b1_ref[...]
        h = jnp.maximum(h, 0.0)
        o_ref[...] = jnp.sum(h * w2_ref[...], axis=1, keepdims=True) + b2_ref[...]

    return pl.pallas_call(
        body,
        out_shape=jax.ShapeDtypeStruct((B, 1), jnp.float32),
    )(pooled_sum, w1t, b1, w2, b2)


def kernel(x, emb, W1, b1, W2, b2):
    table = jnp.concatenate([emb, emb], axis=1)
    pooled_sum = _pool_sc(x, table)
    out = _mlp_tc(
        pooled_sum,
        W1.T,
        b1.reshape(1, H),
        W2.reshape(1, H),
        b2.reshape(1, 1),
    )
    return out.reshape(B)


# speedup vs baseline: 1.2207x; 1.2207x over previous
import functools

import jax
import jax.numpy as jnp
from jax import lax
from jax.experimental import pallas as pl
from jax.experimental.pallas import tpu as pltpu
from jax.experimental.pallas import tpu_sc as plsc

V = 1000000
B = 4096
L = 200
D = 64
H = 32
NC = 2
NS = 16
NW = NC * NS
BPW = B // NW
NACC = 4
NV = D // 16
TB = 2048


def _relayout_tc(emb_t):
    def body(in_ref, o_ref):
        t = in_ref[...].T
        o_ref[:, 0:D] = t
        o_ref[:, D : 2 * D] = t

    return pl.pallas_call(
        body,
        grid=(V // TB,),
        in_specs=[pl.BlockSpec((D, TB), lambda i: (0, i))],
        out_specs=pl.BlockSpec((TB, 2 * D), lambda i: (i, 0)),
        out_shape=jax.ShapeDtypeStruct((V, 2 * D), jnp.float32),
    )(emb_t)


def _pool_sc(x, table):
    mesh = plsc.VectorSubcoreMesh(core_axis_name="core", subcore_axis_name="subcore")

    @functools.partial(
        pl.kernel,
        out_type=jax.ShapeDtypeStruct((B, D), jnp.float32),
        mesh=mesh,
        scratch_types=[
            pltpu.VMEM((BPW, L), jnp.int32),
            pltpu.VMEM((L, BPW), jnp.int32),
            pltpu.VMEM((NACC, BPW, 2 * D), jnp.float32),
            pltpu.VMEM((BPW, D), jnp.float32),
        ]
        + [pltpu.SemaphoreType.DMA] * NACC,
        compiler_params=pltpu.CompilerParams(
            use_tc_tiling_on_sc=False, needs_layout_passes=False
        ),
    )
    def pool(x_hbm, table_hbm, out_hbm, idx_raw, idx_v, accs_v, out_v, *sems):
        wid = lax.axis_index("subcore") * NC + lax.axis_index("core")
        base = wid * BPW
        pltpu.sync_copy(x_hbm.at[pl.ds(base, BPW)], idx_raw)

        lanes = lax.iota(jnp.int32, 16)

        @pl.loop(0, L)
        def _(j):
            cols = jnp.zeros((16,), jnp.int32) + j
            for g in range(BPW // 16):
                v = plsc.load_gather(idx_raw, [lanes + 16 * g, cols])
                idx_v[j, pl.ds(16 * g, 16)] = v

        for k in range(NACC):
            pltpu.async_copy(table_hbm.at[idx_v.at[k]], accs_v.at[k], sems[k])

        @pl.loop(NACC, L, step=NACC)
        def _(j):
            for k in range(NACC):
                pltpu.make_async_copy(
                    table_hbm.at[idx_v.at[0]], accs_v.at[k], sems[k]
                ).wait()
                pltpu.async_copy(
                    table_hbm.at[idx_v.at[j + k]], accs_v.at[k], sems[k], add=True
                )

        for k in range(NACC):
            pltpu.make_async_copy(
                table_hbm.at[idx_v.at[0]], accs_v.at[k], sems[k]
            ).wait()

        @pl.loop(0, BPW)
        def _(b):
            for i in range(NV):
                s = pl.ds(16 * i, 16)
                out_v[b, s] = (
                    (accs_v[0, b, s] + accs_v[1, b, s])
                    + (accs_v[2, b, s] + accs_v[3, b, s])
                )

        pltpu.sync_copy(out_v, out_hbm.at[pl.ds(base, BPW)])

    return pool(x, table)


def _mlp_tc(pooled_sum, w1t, b1, w2, b2):
    def body(p_ref, w1_ref, b1_ref, w2_ref, b2_ref, o_ref):
        p = p_ref[...] * (1.0 / L)
        h = jnp.dot(p, w1_ref[...], preferred_element_type=jnp.float32) + b1_ref[...]
        h = jnp.maximum(h, 0.0)
        o_ref[...] = jnp.sum(h * w2_ref[...], axis=1, keepdims=True) + b2_ref[...]

    return pl.pallas_call(
        body,
        out_shape=jax.ShapeDtypeStruct((B, 1), jnp.float32),
    )(pooled_sum, w1t, b1, w2, b2)


def kernel(x, emb, W1, b1, W2, b2):
    table = _relayout_tc(emb.T)
    pooled_sum = _pool_sc(x, table)
    out = _mlp_tc(
        pooled_sum,
        W1.T,
        b1.reshape(1, H),
        W2.reshape(1, H),
        b2.reshape(1, 1),
    )
    return out.reshape(B)


# speedup vs baseline: 1.2219x; 1.0010x over previous
import functools

import jax
import jax.numpy as jnp
from jax import lax
from jax.experimental import pallas as pl
from jax.experimental.pallas import tpu as pltpu
from jax.experimental.pallas import tpu_sc as plsc

V = 1000000
B = 4096
L = 200
D = 64
H = 32
NC = 2
NS = 16
NW = NC * NS
BPW = B // NW
NACC = 4
NV = D // 16
TB = 2048


def _relayout_tc(emb_t):
    def body(in_ref, o_ref):
        t = in_ref[...].T
        o_ref[:, 0:D] = t
        o_ref[:, D : 2 * D] = t

    return pl.pallas_call(
        body,
        grid=((V + TB - 1) // TB,),
        in_specs=[pl.BlockSpec((D, TB), lambda i: (0, i))],
        out_specs=pl.BlockSpec((TB, 2 * D), lambda i: (i, 0)),
        out_shape=jax.ShapeDtypeStruct((V, 2 * D), jnp.float32),
    )(emb_t)


def _pool_sc(x, table):
    mesh = plsc.VectorSubcoreMesh(core_axis_name="core", subcore_axis_name="subcore")

    @functools.partial(
        pl.kernel,
        out_type=jax.ShapeDtypeStruct((B, D), jnp.float32),
        mesh=mesh,
        scratch_types=[
            pltpu.VMEM((BPW, L), jnp.int32),
            pltpu.VMEM((L, BPW), jnp.int32),
            pltpu.VMEM((NACC, BPW, 2 * D), jnp.float32),
            pltpu.VMEM((BPW, D), jnp.float32),
        ]
        + [pltpu.SemaphoreType.DMA] * NACC,
        compiler_params=pltpu.CompilerParams(
            use_tc_tiling_on_sc=False, needs_layout_passes=False
        ),
    )
    def pool(x_hbm, table_hbm, out_hbm, idx_raw, idx_v, accs_v, out_v, *sems):
        wid = lax.axis_index("subcore") * NC + lax.axis_index("core")
        base = wid * BPW
        pltpu.sync_copy(x_hbm.at[pl.ds(base, BPW)], idx_raw)

        lanes = lax.iota(jnp.int32, 16)

        @pl.loop(0, L)
        def _(j):
            cols = jnp.zeros((16,), jnp.int32) + j
            for g in range(BPW // 16):
                v = plsc.load_gather(idx_raw, [lanes + 16 * g, cols])
                idx_v[j, pl.ds(16 * g, 16)] = v

        for k in range(NACC):
            pltpu.async_copy(table_hbm.at[idx_v.at[k]], accs_v.at[k], sems[k])

        @pl.loop(NACC, L, step=NACC)
        def _(j):
            for k in range(NACC):
                pltpu.make_async_copy(
                    table_hbm.at[idx_v.at[0]], accs_v.at[k], sems[k]
                ).wait()
                pltpu.async_copy(
                    table_hbm.at[idx_v.at[j + k]], accs_v.at[k], sems[k], add=True
                )

        for k in range(NACC):
            pltpu.make_async_copy(
                table_hbm.at[idx_v.at[0]], accs_v.at[k], sems[k]
            ).wait()

        @pl.loop(0, BPW)
        def _(b):
            for i in range(NV):
                s = pl.ds(16 * i, 16)
                out_v[b, s] = (
                    (accs_v[0, b, s] + accs_v[1, b, s])
                    + (accs_v[2, b, s] + accs_v[3, b, s])
                )

        pltpu.sync_copy(out_v, out_hbm.at[pl.ds(base, BPW)])

    return pool(x, table)


def _mlp_tc(pooled_sum, w1t, b1, w2, b2):
    def body(p_ref, w1_ref, b1_ref, w2_ref, b2_ref, o_ref):
        p = p_ref[...] * (1.0 / L)
        h = jnp.dot(p, w1_ref[...], preferred_element_type=jnp.float32) + b1_ref[...]
        h = jnp.maximum(h, 0.0)
        o_ref[...] = jnp.sum(h * w2_ref[...], axis=1, keepdims=True) + b2_ref[...]

    return pl.pallas_call(
        body,
        out_shape=jax.ShapeDtypeStruct((B, 1), jnp.float32),
    )(pooled_sum, w1t, b1, w2, b2)


def kernel(x, emb, W1, b1, W2, b2):
    table = _relayout_tc(emb.T)
    pooled_sum = _pool_sc(x, table)
    out = _mlp_tc(
        pooled_sum,
        W1.T,
        b1.reshape(1, H),
        W2.reshape(1, H),
        b2.reshape(1, 1),
    )
    return out.reshape(B)


# speedup vs baseline: 2.4369x; 1.9944x over previous
import functools

import jax
import jax.numpy as jnp
from jax import lax
from jax.experimental import pallas as pl
from jax.experimental.pallas import tpu as pltpu
from jax.experimental.pallas import tpu_sc as plsc

V = 1000000
B = 4096
L = 200
D = 64
H = 32
NC = 2
NS = 16
NW = NC * NS
BPW = B // NW
NACC = 4
NV = D // 16
TB = 8192


def _relayout_tc(emb_t):
    def body(in_ref, o_ref):
        t = in_ref[...].T
        o_ref[...] = jnp.concatenate([t, t], axis=1)

    return pl.pallas_call(
        body,
        grid=((V + TB - 1) // TB,),
        in_specs=[pl.BlockSpec((D, TB), lambda i: (0, i))],
        out_specs=pl.BlockSpec((TB, 2 * D), lambda i: (i, 0)),
        out_shape=jax.ShapeDtypeStruct((V, 2 * D), jnp.float32),
    )(emb_t)


def _pool_sc(x, table):
    mesh = plsc.VectorSubcoreMesh(core_axis_name="core", subcore_axis_name="subcore")

    @functools.partial(
        pl.kernel,
        out_type=jax.ShapeDtypeStruct((B, D), jnp.float32),
        mesh=mesh,
        scratch_types=[
            pltpu.VMEM((BPW, L), jnp.int32),
            pltpu.VMEM((L, BPW), jnp.int32),
            pltpu.VMEM((NACC, BPW, D), jnp.float32),
            pltpu.VMEM((BPW, D), jnp.float32),
        ]
        + [pltpu.SemaphoreType.DMA] * NACC,
        compiler_params=pltpu.CompilerParams(
            use_tc_tiling_on_sc=False, needs_layout_passes=False
        ),
    )
    def pool(x_hbm, table_hbm, out_hbm, idx_raw, idx_v, accs_v, out_v, *sems):
        wid = lax.axis_index("subcore") * NC + lax.axis_index("core")
        base = wid * BPW
        pltpu.sync_copy(x_hbm.at[pl.ds(base, BPW)], idx_raw)

        lanes = lax.iota(jnp.int32, 16)

        @pl.loop(0, L)
        def _(j):
            cols = jnp.zeros((16,), jnp.int32) + j
            for g in range(BPW // 16):
                v = plsc.load_gather(idx_raw, [lanes + 16 * g, cols])
                idx_v[j, pl.ds(16 * g, 16)] = v + v

        for k in range(NACC):
            pltpu.async_copy(table_hbm.at[idx_v.at[k]], accs_v.at[k], sems[k])

        @pl.loop(NACC, L, step=NACC)
        def _(j):
            for k in range(NACC):
                pltpu.make_async_copy(
                    table_hbm.at[idx_v.at[0]], accs_v.at[k], sems[k]
                ).wait()
                pltpu.async_copy(
                    table_hbm.at[idx_v.at[j + k]], accs_v.at[k], sems[k], add=True
                )

        for k in range(NACC):
            pltpu.make_async_copy(
                table_hbm.at[idx_v.at[0]], accs_v.at[k], sems[k]
            ).wait()

        @pl.loop(0, BPW)
        def _(b):
            for i in range(NV):
                s = pl.ds(16 * i, 16)
                out_v[b, s] = (
                    (accs_v[0, b, s] + accs_v[1, b, s])
                    + (accs_v[2, b, s] + accs_v[3, b, s])
                )

        pltpu.sync_copy(out_v, out_hbm.at[pl.ds(base, BPW)])

    return pool(x, table)


def _mlp_tc(pooled_sum, w1t, b1, w2, b2):
    def body(p_ref, w1_ref, b1_ref, w2_ref, b2_ref, o_ref):
        p = p_ref[...] * (1.0 / L)
        h = jnp.dot(p, w1_ref[...], preferred_element_type=jnp.float32) + b1_ref[...]
        h = jnp.maximum(h, 0.0)
        o_ref[...] = jnp.sum(h * w2_ref[...], axis=1, keepdims=True) + b2_ref[...]

    return pl.pallas_call(
        body,
        out_shape=jax.ShapeDtypeStruct((B, 1), jnp.float32),
    )(pooled_sum, w1t, b1, w2, b2)


def kernel(x, emb, W1, b1, W2, b2):
    table = _relayout_tc(emb.T).reshape(2 * V, D)
    pooled_sum = _pool_sc(x, table)
    out = _mlp_tc(
        pooled_sum,
        W1.T,
        b1.reshape(1, H),
        W2.reshape(1, H),
        b2.reshape(1, 1),
    )
    return out.reshape(B)
